# baseline (speedup 1.0000x reference)
import functools
import numpy as np
import jax
import jax.numpy as jnp
from jax import lax
from jax.experimental import pallas as pl
from jax.experimental.pallas import tpu as pltpu
from jax.experimental.pallas import tpu_sc as plsc

_D_MODEL = 64
_MAX_LEN = 200
_LANES = 16
_NUM_WORKERS = 32


def _pos_encoding_np(position, d_model):
    def get_angles(pos, i, d_model):
        angle_rates = 1 / np.power(10000, 2 * (i // 2) / np.float32(d_model))
        return pos * angle_rates

    angle_rads = get_angles(np.arange(position)[:, np.newaxis],
                            np.arange(d_model)[np.newaxis, :], d_model)
    angle_rads[:, 0::2] = np.sin(angle_rads[:, 0::2])
    angle_rads[:, 1::2] = np.cos(angle_rads[:, 1::2])
    return angle_rads.astype(np.float32)


@functools.lru_cache(maxsize=None)
def _build_kernel(batch, seq, vocab):
    assert batch % _NUM_WORKERS == 0
    seq_per_w = batch // _NUM_WORKERS
    n_tokens = batch * seq
    scale = float(np.sqrt(np.float32(_D_MODEL)))

    mesh = plsc.VectorSubcoreMesh(core_axis_name="c", subcore_axis_name="s")

    @functools.partial(
        pl.kernel,
        mesh=mesh,
        out_type=jax.ShapeDtypeStruct((n_tokens, _D_MODEL), jnp.float32),
        scratch_types=[
            pltpu.VMEM((seq,), jnp.int32),
            pltpu.VMEM((seq, _D_MODEL), jnp.float32),
            pltpu.VMEM((seq, _D_MODEL), jnp.float32),
            pltpu.SemaphoreType.DMA,
        ],
        compiler_params=pltpu.CompilerParams(use_tc_tiling_on_sc=False),
    )
    def emb(idx_hbm, table_hbm, pos_hbm, out_hbm, idx_v, buf_v, pos_v, sem):
        wid = lax.axis_index("s") * 2 + lax.axis_index("c")
        pltpu.sync_copy(pos_hbm, pos_v)

        def seq_body(k, carry):
            base = (wid * seq_per_w + k) * seq
            pltpu.sync_copy(idx_hbm.at[pl.ds(base, seq)], idx_v)
            pltpu.async_copy(table_hbm.at[idx_v], buf_v, sem).wait()

            def row_body(r, c2):
                for c in range(_D_MODEL // _LANES):
                    sl = pl.ds(c * _LANES, _LANES)
                    buf_v[r, sl] = buf_v[r, sl] * scale + pos_v[r, sl]
                return c2

            lax.fori_loop(0, seq, row_body, 0)
            pltpu.sync_copy(buf_v, out_hbm.at[pl.ds(base, seq)])
            return carry

        lax.fori_loop(0, seq_per_w, seq_body, 0)

    return emb


def kernel(inputs, table):
    batch, seq = inputs.shape
    vocab = table.shape[0]
    pos = jnp.asarray(_pos_encoding_np(_MAX_LEN, _D_MODEL)[:seq])
    emb = _build_kernel(batch, seq, vocab)
    out = emb(inputs.reshape(-1), table, pos)
    return out.reshape(batch, seq, _D_MODEL)


# speedup vs baseline: 1.2213x; 1.2213x over previous
import functools
import numpy as np
import jax
import jax.numpy as jnp
from jax import lax
from jax.experimental import pallas as pl
from jax.experimental.pallas import tpu as pltpu
from jax.experimental.pallas import tpu_sc as plsc

_D_MODEL = 64
_MAX_LEN = 200
_LANES = 16
_NUM_WORKERS = 32


def _pos_encoding_np(position, d_model):
    def get_angles(pos, i, d_model):
        angle_rates = 1 / np.power(10000, 2 * (i // 2) / np.float32(d_model))
        return pos * angle_rates

    angle_rads = get_angles(np.arange(position)[:, np.newaxis],
                            np.arange(d_model)[np.newaxis, :], d_model)
    angle_rads[:, 0::2] = np.sin(angle_rads[:, 0::2])
    angle_rads[:, 1::2] = np.cos(angle_rads[:, 1::2])
    return angle_rads.astype(np.float32)


@functools.lru_cache(maxsize=None)
def _build_kernel(batch, seq, vocab):
    assert batch % _NUM_WORKERS == 0
    seq_per_w = batch // _NUM_WORKERS
    tok_per_w = seq_per_w * seq
    n_tokens = batch * seq
    scale = float(np.sqrt(np.float32(_D_MODEL)))
    nbuf = 2

    mesh = plsc.VectorSubcoreMesh(core_axis_name="c", subcore_axis_name="s")

    @functools.partial(
        pl.kernel,
        mesh=mesh,
        out_type=jax.ShapeDtypeStruct((n_tokens, _D_MODEL), jnp.float32),
        scratch_types=[
            pltpu.VMEM((tok_per_w,), jnp.int32),
            pltpu.VMEM((seq, _D_MODEL), jnp.float32),
            [pltpu.VMEM((seq, _D_MODEL), jnp.float32)] * nbuf,
            [pltpu.VMEM((seq, _D_MODEL), jnp.float32)] * nbuf,
            [pltpu.SemaphoreType.DMA] * nbuf,
            [pltpu.SemaphoreType.DMA] * nbuf,
        ],
        compiler_params=pltpu.CompilerParams(use_tc_tiling_on_sc=False),
    )
    def emb(idx_hbm, table_hbm, pos_hbm, out_hbm,
            idx_v, pos_v, gbuf, obuf, semg, semo):
        wid = lax.axis_index("s") * 2 + lax.axis_index("c")
        wbase = wid * tok_per_w
        pltpu.sync_copy(pos_hbm, pos_v)
        pltpu.sync_copy(idx_hbm.at[pl.ds(wbase, tok_per_w)], idx_v)

        def start_gather(g, b):
            return pltpu.async_copy(
                table_hbm.at[idx_v.at[pl.ds(g * seq, seq)]], gbuf[b], semg[b])

        def wait_gather(g, b):
            pltpu.make_async_copy(
                table_hbm.at[idx_v.at[pl.ds(g * seq, seq)]], gbuf[b],
                semg[b]).wait()

        def start_out(g, b):
            return pltpu.async_copy(
                obuf[b], out_hbm.at[pl.ds(wbase + g * seq, seq)], semo[b])

        def wait_out(g, b):
            pltpu.make_async_copy(
                obuf[b], out_hbm.at[pl.ds(wbase + g * seq, seq)],
                semo[b]).wait()

        for b in range(nbuf):
            start_gather(b, b)

        def chunk_round(k, carry):
            for b in range(nbuf):
                g = k + b
                wait_gather(g, b)

                @pl.when(g >= nbuf)
                def _():
                    wait_out(g - nbuf, b)

                def row_body(r, c2):
                    for c in range(_D_MODEL // _LANES):
                        sl = pl.ds(c * _LANES, _LANES)
                        obuf[b][r, sl] = gbuf[b][r, sl] * scale + pos_v[r, sl]
                    return c2

                lax.fori_loop(0, seq, row_body, 0)
                start_out(g, b)

                @pl.when(g + nbuf < seq_per_w)
                def _():
                    start_gather(g + nbuf, b)
            return carry

        lax.fori_loop(0, seq_per_w // nbuf, lambda k, c: chunk_round(k * nbuf, c), 0)
        for b in range(nbuf):
            wait_out(seq_per_w - nbuf + b, b)

    return emb


def kernel(inputs, table):
    batch, seq = inputs.shape
    vocab = table.shape[0]
    pos = jnp.asarray(_pos_encoding_np(_MAX_LEN, _D_MODEL)[:seq])
    emb = _build_kernel(batch, seq, vocab)
    out = emb(inputs.reshape(-1), table, pos)
    return out.reshape(batch, seq, _D_MODEL)
